# Initial kernel scaffold; baseline (speedup 1.0000x reference)
import jax
import jax.numpy as jnp
from jax.experimental import pallas as pl


def kernel(x, edge_index, W1, b1, W2, b2, W3, b3):
    raise NotImplementedError("write your pallas kernel here")


# speedup vs baseline: 29.4919x; 29.4919x over previous
import functools

import jax
import jax.numpy as jnp
from jax import lax
from jax.experimental import pallas as pl
from jax.experimental.pallas import tpu as pltpu
from jax.experimental.pallas import tpu_sc as plsc

N = 50000
E = 800000
NW = 32
CH = 128
NCH = 196
EPW = CH * NCH
EPAD = NW * EPW
NP = 50176
RPT = NP // 16
NOUT = 50048
CPT = NOUT // 16
KB = 14
BN = 2000
GRID = N // BN

_mesh = plsc.VectorSubcoreMesh(core_axis_name="c", subcore_axis_name="s")


@functools.partial(
    pl.kernel, mesh=_mesh,
    compiler_params=pltpu.CompilerParams(use_tc_tiling_on_sc=False),
    out_type=jax.ShapeDtypeStruct((2, NOUT, 32), jnp.float32),
    scratch_types=[
        pltpu.VMEM((KB, 2, CH), jnp.int32),
        pltpu.VMEM((CH, 32), jnp.float32),
        pltpu.VMEM((CH, 32), jnp.float32),
        pltpu.VMEM_SHARED((NP, 32), jnp.float32),
        pltpu.SemaphoreType.DMA,
        pltpu.SemaphoreType.DMA,
    ],
)
def _sc_agg(g_hbm, idx_hbm, z_hbm, out_hbm,
            idx_v, rows0, rows1, acc, sem0, sem1):
    c = lax.axis_index("c")
    s = lax.axis_index("s")
    w = s * 2 + c

    def zinit(i, carry):
        pltpu.sync_copy(z_hbm, acc.at[pl.ds(s * RPT + i * 392, 392)])
        return carry
    lax.fori_loop(0, RPT // 392, zinit, 0)
    plsc.subcore_barrier()

    for b in range(NCH // KB):
        pltpu.sync_copy(idx_hbm.at[w, pl.ds(b * KB, KB)], idx_v)
        pltpu.make_async_copy(g_hbm.at[idx_v.at[0, 0]], rows0, sem0).start()
        pltpu.make_async_copy(g_hbm.at[idx_v.at[1, 0]], rows1, sem1).start()

        def step(jj, carry):
            j0 = jj * 2
            pltpu.make_async_copy(g_hbm.at[idx_v.at[j0, 0]], rows0,
                                  sem0).wait()
            pltpu.sync_copy(rows0, acc.at[idx_v.at[j0, 1]], add=True)

            @pl.when(jj < KB // 2 - 1)
            def _():
                pltpu.make_async_copy(g_hbm.at[idx_v.at[j0 + 2, 0]], rows0,
                                      sem0).start()

            j1 = j0 + 1
            pltpu.make_async_copy(g_hbm.at[idx_v.at[j1, 0]], rows1,
                                  sem1).wait()
            pltpu.sync_copy(rows1, acc.at[idx_v.at[j1, 1]], add=True)

            @pl.when(jj < KB // 2 - 1)
            def _():
                pltpu.make_async_copy(g_hbm.at[idx_v.at[j1 + 2, 0]], rows1,
                                      sem1).start()
            return carry
        lax.fori_loop(0, KB // 2, step, 0)

    plsc.subcore_barrier()

    pltpu.sync_copy(acc.at[pl.ds(s * CPT, CPT)],
                    out_hbm.at[c, pl.ds(s * CPT, CPT)])


@functools.partial(
    pl.kernel, mesh=_mesh,
    compiler_params=pltpu.CompilerParams(use_tc_tiling_on_sc=False),
    out_type=jax.ShapeDtypeStruct((2, NOUT, 8), jnp.float32),
    scratch_types=[
        pltpu.VMEM((NCH, CH), jnp.int32),
        pltpu.VMEM((CH, 8), jnp.float32),
        pltpu.VMEM((64, 8), jnp.float32),
        pltpu.VMEM_SHARED((NP, 8), jnp.float32),
    ],
)
def _sc_deg(dst_hbm, z_hbm, ones_hbm, out_hbm, dst_v, ones_v, zv, acc):
    c = lax.axis_index("c")
    s = lax.axis_index("s")
    w = s * 2 + c

    pltpu.sync_copy(z_hbm, zv)

    def zinit(i, carry):
        pltpu.sync_copy(zv, acc.at[pl.ds(s * RPT + i * 64, 64)])
        return carry
    lax.fori_loop(0, RPT // 64, zinit, 0)

    pltpu.sync_copy(dst_hbm.at[w], dst_v)
    pltpu.sync_copy(ones_hbm, ones_v)
    plsc.subcore_barrier()

    def step(j, carry):
        pltpu.sync_copy(ones_v, acc.at[dst_v.at[j]], add=True)
        return carry
    lax.fori_loop(0, NCH, step, 0)

    plsc.subcore_barrier()
    pltpu.sync_copy(acc.at[pl.ds(s * CPT, CPT)],
                    out_hbm.at[c, pl.ds(s * CPT, CPT)])


def _celu(t):
    return jnp.where(t > 0, t, jnp.exp(t) - 1.0)


def _tc1_body(x_ref, w1_ref, deg_ref, dinv_ref, g1_ref):
    deg = deg_ref[0, :, 0:1] + deg_ref[1, :, 0:1] + 1.0
    di = lax.rsqrt(deg)
    dinv_ref[...] = di
    g1_ref[...] = di * jnp.dot(x_ref[...], w1_ref[...],
                               preferred_element_type=jnp.float32)


def _tc2_body(s_ref, g1_ref, dinv_ref, m1_ref, b1_ref, g2_ref):
    di = dinv_ref[...]
    t = di * (s_ref[0] + s_ref[1] + g1_ref[...]) + b1_ref[...]
    g2_ref[...] = di * (2.0 * m1_ref[...] * _celu(t))


def _tc3_body(s_ref, g2_ref, dinv_ref, m2_ref, w2_ref, b2_ref, w3_ref,
              g3_ref):
    di = dinv_ref[...]
    u = di * (s_ref[0] + s_ref[1] + g2_ref[...])
    h2 = _celu(jnp.dot(u, w2_ref[...], preferred_element_type=jnp.float32)
               + b2_ref[...])
    d2 = 2.0 * m2_ref[...] * h2
    g3_ref[...] = di * jnp.dot(d2, w3_ref[...],
                               preferred_element_type=jnp.float32)


def _tc4_body(s_ref, g3_ref, dinv_ref, b3_ref, out_ref):
    di = dinv_ref[...]
    t = di * (s_ref[0] + s_ref[1] + g3_ref[...]) + b3_ref[...]
    out_ref[...] = _celu(t)


def _rows(width):
    return pl.BlockSpec((BN, width), lambda i: (i, 0))


def _pair(width):
    return pl.BlockSpec((2, BN, width), lambda i: (0, i, 0))


def _whole(shape):
    return pl.BlockSpec(shape, lambda i: tuple(0 for _ in shape))


def _tc1(x, W1, degp):
    return pl.pallas_call(
        _tc1_body,
        grid=(GRID,),
        in_specs=[_rows(64), _whole((64, 32)), _pair(8)],
        out_specs=[_rows(1), _rows(32)],
        out_shape=[jax.ShapeDtypeStruct((N, 1), jnp.float32),
                   jax.ShapeDtypeStruct((N, 32), jnp.float32)],
    )(x, W1, degp)


def _tc2(s1, g1, dinv, m1, b1):
    return pl.pallas_call(
        _tc2_body,
        grid=(GRID,),
        in_specs=[_pair(32), _rows(32), _rows(1), _rows(32),
                  _whole((1, 32))],
        out_specs=_rows(32),
        out_shape=jax.ShapeDtypeStruct((N, 32), jnp.float32),
    )(s1, g1, dinv, m1, b1)


def _tc3(s2, g2, dinv, m2, W2, b2, W3):
    return pl.pallas_call(
        _tc3_body,
        grid=(GRID,),
        in_specs=[_pair(32), _rows(32), _rows(1), _rows(64),
                  _whole((32, 64)), _whole((1, 64)), _whole((64, 32))],
        out_specs=_rows(32),
        out_shape=jax.ShapeDtypeStruct((N, 32), jnp.float32),
    )(s2, g2, dinv, m2, W2, b2, W3)


def _tc4(s3, g3, dinv, b3):
    return pl.pallas_call(
        _tc4_body,
        grid=(GRID,),
        in_specs=[_pair(32), _rows(32), _rows(1), _whole((1, 32))],
        out_specs=_rows(32),
        out_shape=jax.ShapeDtypeStruct((N, 32), jnp.float32),
    )(s3, g3, dinv, b3)


def kernel(x, edge_index, W1, b1, W2, b2, W3, b3):
    src = edge_index[0]
    dst = edge_index[1]
    pad = EPAD - E
    srcp = jnp.concatenate(
        [src, jnp.zeros((pad,), jnp.int32)]).reshape(NW, NCH, CH)
    dstp = jnp.concatenate(
        [dst, jnp.full((pad,), N, jnp.int32)]).reshape(NW, NCH, CH)
    idxp = jnp.stack([srcp, dstp], axis=2)

    z32 = jnp.zeros((392, 32), jnp.float32)
    z8 = jnp.zeros((64, 8), jnp.float32)
    ones8 = jnp.ones((CH, 8), jnp.float32)

    dk = jax.random.key(42)
    k1, k2 = jax.random.split(dk)
    m1 = jax.random.bernoulli(k1, 0.5, (N, 32)).astype(jnp.float32)
    m2 = jax.random.bernoulli(k2, 0.5, (N, 64)).astype(jnp.float32)

    degp = _sc_deg(dstp, z8, ones8)
    dinv, g1 = _tc1(x, W1, degp)

    s1 = _sc_agg(g1, idxp, z32)
    g2 = _tc2(s1, g1, dinv, m1, b1.reshape(1, 32))

    s2 = _sc_agg(g2, idxp, z32)
    g3 = _tc3(s2, g2, dinv, m2, W2, b2.reshape(1, 64), W3)

    s3 = _sc_agg(g3, idxp, z32)
    return _tc4(s3, g3, dinv, b3.reshape(1, 32))


# speedup vs baseline: 35.9065x; 1.2175x over previous
import functools

import jax
import jax.numpy as jnp
from jax import lax
from jax.experimental import pallas as pl
from jax.experimental.pallas import tpu as pltpu
from jax.experimental.pallas import tpu_sc as plsc

N = 50000
E = 800000
NW = 32
CH = 128
NCH = 196
KB = 14
EPW = CH * NCH
EPAD = NW * EPW
NP = 50176
RPT = NP // 16
NOUT = 50048
CPT = NOUT // 16
NPK = N // 4
NPP = NOUT // 4
BP = 3128
GRIDP = NPP // BP

_mesh = plsc.VectorSubcoreMesh(core_axis_name="c", subcore_axis_name="s")


@functools.partial(
    pl.kernel, mesh=_mesh,
    compiler_params=pltpu.CompilerParams(use_tc_tiling_on_sc=False),
    out_type=jax.ShapeDtypeStruct((2, NOUT, 32), jnp.float32),
    scratch_types=[
        pltpu.VMEM((KB, CH), jnp.int32),
        pltpu.VMEM((KB, CH), jnp.int32),
        pltpu.VMEM((CH, 32), jnp.float32),
        pltpu.VMEM((CH, 32), jnp.float32),
        pltpu.VMEM_SHARED((NP, 32), jnp.float32),
        pltpu.SemaphoreType.DMA,
        pltpu.SemaphoreType.DMA,
    ],
)
def _sc_agg(g_hbm, src_hbm, dst_hbm, z_hbm, out_hbm,
            src_v, dst_v, rows0, rows1, acc, sem0, sem1):
    c = lax.axis_index("c")
    s = lax.axis_index("s")
    w = s * 2 + c

    def zinit(i, carry):
        pltpu.sync_copy(z_hbm, acc.at[pl.ds(s * RPT + i * 392, 392)])
        return carry
    lax.fori_loop(0, RPT // 392, zinit, 0)
    plsc.subcore_barrier()

    for b in range(NCH // KB):
        pltpu.sync_copy(src_hbm.at[w, pl.ds(b * KB, KB)], src_v)
        pltpu.sync_copy(dst_hbm.at[w, pl.ds(b * KB, KB)], dst_v)
        pltpu.make_async_copy(g_hbm.at[src_v.at[0]], rows0, sem0).start()
        pltpu.make_async_copy(g_hbm.at[src_v.at[1]], rows1, sem1).start()

        def step(jj, carry):
            j0 = jj * 2
            pltpu.make_async_copy(g_hbm.at[src_v.at[j0]], rows0,
                                  sem0).wait()
            pltpu.sync_copy(rows0, acc.at[dst_v.at[j0]], add=True)

            @pl.when(jj < KB // 2 - 1)
            def _():
                pltpu.make_async_copy(g_hbm.at[src_v.at[j0 + 2]], rows0,
                                      sem0).start()

            j1 = j0 + 1
            pltpu.make_async_copy(g_hbm.at[src_v.at[j1]], rows1,
                                  sem1).wait()
            pltpu.sync_copy(rows1, acc.at[dst_v.at[j1]], add=True)

            @pl.when(jj < KB // 2 - 1)
            def _():
                pltpu.make_async_copy(g_hbm.at[src_v.at[j1 + 2]], rows1,
                                      sem1).start()
            return carry
        lax.fori_loop(0, KB // 2, step, 0)

    plsc.subcore_barrier()

    pltpu.sync_copy(acc.at[pl.ds(s * CPT, CPT)],
                    out_hbm.at[c, pl.ds(s * CPT, CPT)])


@functools.partial(
    pl.kernel, mesh=_mesh,
    compiler_params=pltpu.CompilerParams(use_tc_tiling_on_sc=False),
    out_type=jax.ShapeDtypeStruct((2, NOUT, 32), jnp.float32),
    scratch_types=[
        pltpu.VMEM((KB, CH), jnp.int32),
        pltpu.VMEM((CH, 32), jnp.float32),
        pltpu.VMEM_SHARED((NP, 32), jnp.float32),
    ],
)
def _sc_deg(dst_hbm, z_hbm, ones_hbm, out_hbm, dst_v, ones_v, acc):
    c = lax.axis_index("c")
    s = lax.axis_index("s")
    w = s * 2 + c

    def zinit(i, carry):
        pltpu.sync_copy(z_hbm, acc.at[pl.ds(s * RPT + i * 392, 392)])
        return carry
    lax.fori_loop(0, RPT // 392, zinit, 0)
    pltpu.sync_copy(ones_hbm, ones_v)
    plsc.subcore_barrier()

    for b in range(NCH // KB):
        pltpu.sync_copy(dst_hbm.at[w, pl.ds(b * KB, KB)], dst_v)

        def step(j, carry):
            pltpu.sync_copy(ones_v, acc.at[dst_v.at[j]], add=True)
            return carry
        lax.fori_loop(0, KB, step, 0)

    plsc.subcore_barrier()
    pltpu.sync_copy(acc.at[pl.ds(s * CPT, CPT)],
                    out_hbm.at[c, pl.ds(s * CPT, CPT)])


def _celu(t):
    return jnp.where(t > 0, t, jnp.exp(t) - 1.0)


def _tc1_body(x_ref, w1_ref, deg_ref, dinv_ref, g1_ref):
    di = lax.rsqrt(deg_ref[0] + deg_ref[1] + 1.0)
    dinv_ref[...] = di
    g1_ref[...] = di * jnp.dot(x_ref[...], w1_ref[...],
                               preferred_element_type=jnp.float32)


def _tc2_body(s_ref, g1_ref, dinv_ref, m1_ref, b1_ref, g2_ref):
    di = dinv_ref[...]
    t = di * (s_ref[0] + s_ref[1] + g1_ref[...]) + b1_ref[...]
    g2_ref[...] = di * (2.0 * m1_ref[...] * _celu(t))


def _tc3_body(s_ref, g2_ref, dinv_ref, m2_ref, w2_ref, b2_ref, w3_ref,
              g3_ref):
    di = dinv_ref[...]
    u = di * (s_ref[0] + s_ref[1] + g2_ref[...])
    h2 = _celu(jnp.dot(u, w2_ref[...], preferred_element_type=jnp.float32)
               + b2_ref[...])
    d2 = 2.0 * m2_ref[...] * h2
    g3_ref[...] = di * jnp.dot(d2, w3_ref[...],
                               preferred_element_type=jnp.float32)


def _tc4_body(s_ref, g3_ref, dinv_ref, b3_ref, out_ref):
    di = dinv_ref[...]
    t = di * (s_ref[0] + s_ref[1] + g3_ref[...]) + b3_ref[...]
    out_ref[...] = _celu(t)


def _rows(width):
    return pl.BlockSpec((BP, width), lambda i: (i, 0))


def _pair(width):
    return pl.BlockSpec((2, BP, width), lambda i: (0, i, 0))


def _whole(shape):
    return pl.BlockSpec(shape, lambda i: tuple(0 for _ in shape))


def _tc1(xp, W1b, degp):
    return pl.pallas_call(
        _tc1_body,
        grid=(GRIDP,),
        in_specs=[_rows(256), _whole((256, 128)), _pair(128)],
        out_specs=[_rows(128), _rows(128)],
        out_shape=[jax.ShapeDtypeStruct((NPP, 128), jnp.float32),
                   jax.ShapeDtypeStruct((NPP, 128), jnp.float32)],
    )(xp, W1b, degp)


def _tc2(s1, g1, dinv, m1, b1t):
    return pl.pallas_call(
        _tc2_body,
        grid=(GRIDP,),
        in_specs=[_pair(128), _rows(128), _rows(128), _rows(128),
                  _whole((1, 128))],
        out_specs=_rows(128),
        out_shape=jax.ShapeDtypeStruct((NPP, 128), jnp.float32),
    )(s1, g1, dinv, m1, b1t)


def _tc3(s2, g2, dinv, m2, W2b, b2t, W3b):
    return pl.pallas_call(
        _tc3_body,
        grid=(GRIDP,),
        in_specs=[_pair(128), _rows(128), _rows(128), _rows(256),
                  _whole((128, 256)), _whole((1, 256)), _whole((256, 128))],
        out_specs=_rows(128),
        out_shape=jax.ShapeDtypeStruct((NPP, 128), jnp.float32),
    )(s2, g2, dinv, m2, W2b, b2t, W3b)


def _tc4(s3, g3, dinv, b3t):
    return pl.pallas_call(
        _tc4_body,
        grid=(GRIDP,),
        in_specs=[_pair(128), _rows(128), _rows(128), _whole((1, 128))],
        out_specs=_rows(128),
        out_shape=jax.ShapeDtypeStruct((NPP, 128), jnp.float32),
    )(s3, g3, dinv, b3t)


def _bd4(W):
    return jnp.kron(jnp.eye(4, dtype=jnp.float32), W)


def kernel(x, edge_index, W1, b1, W2, b2, W3, b3):
    src = edge_index[0]
    dst = edge_index[1]
    pad = EPAD - E
    srcp = jnp.concatenate(
        [src, jnp.zeros((pad,), jnp.int32)]).reshape(NW, NCH, CH)
    dstp = jnp.concatenate(
        [dst, jnp.full((pad,), N, jnp.int32)]).reshape(NW, NCH, CH)

    z32 = jnp.zeros((392, 32), jnp.float32)
    ones32 = jnp.ones((CH, 32), jnp.float32)

    k1, k2 = jax.random.split(jax.random.key(42))
    padp = ((0, NPP - NPK), (0, 0))
    m1 = jnp.pad(jax.random.bernoulli(k1, 0.5, (N, 32)).astype(
        jnp.float32).reshape(NPK, 128), padp)
    m2 = jnp.pad(jax.random.bernoulli(k2, 0.5, (N, 64)).astype(
        jnp.float32).reshape(NPK, 256), padp)

    xp = jnp.pad(x.reshape(NPK, 256), padp)
    W1b = _bd4(W1)
    W2b = _bd4(W2)
    W3b = _bd4(W3)
    b1t = jnp.tile(b1, 4)[None]
    b2t = jnp.tile(b2, 4)[None]
    b3t = jnp.tile(b3, 4)[None]

    degp = _sc_deg(dstp, z32, ones32).reshape(2, NPP, 128)
    dinv, g1 = _tc1(xp, W1b, degp)

    s1 = _sc_agg(g1.reshape(NOUT, 32), srcp, dstp, z32).reshape(2, NPP, 128)
    g2 = _tc2(s1, g1, dinv, m1, b1t)

    s2 = _sc_agg(g2.reshape(NOUT, 32), srcp, dstp, z32).reshape(2, NPP, 128)
    g3 = _tc3(s2, g2, dinv, m2, W2b, b2t, W3b)

    s3 = _sc_agg(g3.reshape(NOUT, 32), srcp, dstp, z32).reshape(2, NPP, 128)
    return _tc4(s3, g3, dinv, b3t)[:NPK].reshape(N, 32)
